# Initial kernel scaffold; baseline (speedup 1.0000x reference)
import jax
import jax.numpy as jnp
from jax.experimental import pallas as pl


def kernel(x, edge_index, W, att_src, att_dst, bias, phase, ln_scale, ln_bias):
    raise NotImplementedError("write your pallas kernel here")


# speedup vs baseline: 31.4574x; 31.4574x over previous
import functools

import jax
import jax.numpy as jnp
from jax import lax
from jax.experimental import pallas as pl
from jax.experimental.pallas import tpu as pltpu
from jax.experimental.pallas import tpu_sc as plsc

N = 10000
D = 128
H = 4
C = 32
E = 320000

NC = 2
NS = 16
NW = NC * NS
EPW = E // NW
K = 80
NBLK = EPW // K
ROW = 144
AW = 16
RPT = 624
TAIL = N - NS * RPT

_SQRT2 = 1.4142135623730951


def _pre_body(x_ref, w_ref, a_ref, xp_ref, av_ref, as_ref, ad_ref):
    xp = jnp.dot(x_ref[...], w_ref[...], preferred_element_type=jnp.float32)
    xp_ref[...] = xp
    av = jnp.dot(xp, a_ref[...], preferred_element_type=jnp.float32)
    av_ref[...] = av
    zpad = jnp.zeros((av.shape[0], AW - H), jnp.float32)
    as_ref[...] = jnp.concatenate([av[:, :H], zpad], axis=1)
    ad_ref[...] = jnp.concatenate([av[:, H:], zpad], axis=1)


def _tc_pre(x, W, A):
    RB = 1000
    return pl.pallas_call(
        _pre_body,
        grid=(N // RB,),
        in_specs=[
            pl.BlockSpec((RB, D), lambda i: (i, 0)),
            pl.BlockSpec((D, D), lambda i: (0, 0)),
            pl.BlockSpec((D, 2 * H), lambda i: (0, 0)),
        ],
        out_specs=[
            pl.BlockSpec((RB, D), lambda i: (i, 0)),
            pl.BlockSpec((RB, 2 * H), lambda i: (i, 0)),
            pl.BlockSpec((RB, AW), lambda i: (i, 0)),
            pl.BlockSpec((RB, AW), lambda i: (i, 0)),
        ],
        out_shape=[
            jax.ShapeDtypeStruct((N, D), jnp.float32),
            jax.ShapeDtypeStruct((N, 2 * H), jnp.float32),
            jax.ShapeDtypeStruct((N, AW), jnp.float32),
            jax.ShapeDtypeStruct((N, AW), jnp.float32),
        ],
    )(x, W, A)


def _sc_body(xp_hbm, as_hbm, ad_hbm, src_hbm, dst_hbm, out_hbm,
             src_v, dst_v, gs_v, gd_v, rows_v, sbuf_v, acc_sh,
             sem1, sem2, sem3):
    cid = lax.axis_index("c")
    sid = lax.axis_index("s")
    wid = cid * NS + sid
    base_e = wid * EPW

    def _zrow(j, _):
        for cc in range(ROW // 16):
            sbuf_v[j, pl.ds(cc * 16, 16)] = jnp.zeros((16,), jnp.float32)
        return 0
    lax.fori_loop(0, K, _zrow, 0)

    def _zacc(i, _):
        pltpu.sync_copy(sbuf_v.at[pl.ds(0, 48)],
                        acc_sh.at[pl.ds(sid * RPT + i * 48, 48)])
        return 0
    lax.fori_loop(0, RPT // 48, _zacc, 0)

    @pl.when(sid == NS - 1)
    def _ztail():
        pltpu.sync_copy(sbuf_v.at[pl.ds(0, TAIL)],
                        acc_sh.at[pl.ds(NS * RPT, TAIL)])
    plsc.subcore_barrier()

    lane = lax.broadcasted_iota(jnp.int32, (16,), 0)
    headmask = jnp.where(lane < H, 1.0, 0.0).astype(jnp.float32)

    def _blk(i, _):
        eb = base_e + i * K
        pltpu.sync_copy(src_hbm.at[pl.ds(eb, K)], src_v)
        pltpu.sync_copy(dst_hbm.at[pl.ds(eb, K)], dst_v)
        cp1 = pltpu.async_copy(xp_hbm.at[src_v], rows_v, sem1)
        cp2 = pltpu.async_copy(as_hbm.at[src_v], gs_v, sem2)
        cp3 = pltpu.async_copy(ad_hbm.at[dst_v], gd_v, sem3)
        cp1.wait()
        cp2.wait()
        cp3.wait()

        def _edge(j, _):
            al = gs_v[j, pl.ds(0, AW)] + gd_v[j, pl.ds(0, AW)]
            al = jnp.maximum(al, 0.2 * al)
            w = jnp.exp(al)
            sbuf_v[j, pl.ds(D, 16)] = w * headmask
            for c8 in range(D // 16):
                wsc = w[c8 // 2]
                v = rows_v[j, pl.ds(c8 * 16, 16)]
                sbuf_v[j, pl.ds(c8 * 16, 16)] = v * wsc
            return 0
        lax.fori_loop(0, K, _edge, 0)

        pltpu.sync_copy(sbuf_v, acc_sh.at[dst_v], add=True)
        return 0

    lax.fori_loop(0, NBLK, _blk, 0)
    plsc.subcore_barrier()

    pltpu.sync_copy(acc_sh.at[pl.ds(sid * RPT, RPT)],
                    out_hbm.at[cid, pl.ds(sid * RPT, RPT)])

    @pl.when(sid == NS - 1)
    def _ftail():
        pltpu.sync_copy(acc_sh.at[pl.ds(NS * RPT, TAIL)],
                        out_hbm.at[cid, pl.ds(NS * RPT, TAIL)])


def _sc_scatter(xp, asrc_t, adst_t, src, dst):
    mesh = plsc.VectorSubcoreMesh(core_axis_name="c", subcore_axis_name="s")
    f = functools.partial(
        pl.kernel,
        out_type=jax.ShapeDtypeStruct((NC, N, ROW), jnp.float32),
        mesh=mesh,
        compiler_params=pltpu.CompilerParams(needs_layout_passes=False,
                                             use_tc_tiling_on_sc=False),
        scratch_types=[
            pltpu.VMEM((K,), jnp.int32),
            pltpu.VMEM((K,), jnp.int32),
            pltpu.VMEM((K, AW), jnp.float32),
            pltpu.VMEM((K, AW), jnp.float32),
            pltpu.VMEM((K, D), jnp.float32),
            pltpu.VMEM((K, ROW), jnp.float32),
            pltpu.VMEM_SHARED((N, ROW), jnp.float32),
            pltpu.SemaphoreType.DMA,
            pltpu.SemaphoreType.DMA,
            pltpu.SemaphoreType.DMA,
        ],
    )(_sc_body)
    return f(xp, asrc_t, adst_t, src, dst)


def _post_body(acc_ref, xp_ref, av_ref, bias_ref, lns_ref, lnb_ref, o_ref):
    xp = xp_ref[...]
    av = av_ref[...]
    accs = acc_ref[0] + acc_ref[1]
    pieces = []
    for h in range(H):
        a_self = av[:, h:h + 1] + av[:, H + h:H + h + 1]
        w_self = jnp.exp(jnp.maximum(a_self, 0.2 * a_self))
        den = accs[:, D + h:D + h + 1] + w_self
        numh = accs[:, C * h:C * h + C] + xp[:, C * h:C * h + C] * w_self
        pieces.append(numh / den)
    hv = jnp.concatenate(pieces, axis=1) + bias_ref[...]
    hv = jnp.sqrt(hv * hv + 1e-12)
    mu = jnp.mean(hv, axis=1, keepdims=True)
    var = jnp.mean((hv - mu) ** 2, axis=1, keepdims=True)
    hv = (hv - mu) * lax.rsqrt(var + 1e-5) * lns_ref[...] + lnb_ref[...]
    o_ref[...] = hv * 0.5 * (1.0 + lax.erf(hv / _SQRT2))


def _tc_post(acc, xp, av, bias, ln_scale, ln_bias):
    RB = 1000
    return pl.pallas_call(
        _post_body,
        grid=(N // RB,),
        in_specs=[
            pl.BlockSpec((NC, RB, ROW), lambda i: (0, i, 0)),
            pl.BlockSpec((RB, D), lambda i: (i, 0)),
            pl.BlockSpec((RB, 2 * H), lambda i: (i, 0)),
            pl.BlockSpec((1, D), lambda i: (0, 0)),
            pl.BlockSpec((1, D), lambda i: (0, 0)),
            pl.BlockSpec((1, D), lambda i: (0, 0)),
        ],
        out_specs=pl.BlockSpec((RB, D), lambda i: (i, 0)),
        out_shape=jax.ShapeDtypeStruct((N, D), jnp.float32),
    )(acc, xp, av, bias, ln_scale, ln_bias)


def kernel(x, edge_index, W, att_src, att_dst, bias, phase, ln_scale, ln_bias):
    del phase
    src = edge_index[0].astype(jnp.int32)
    dst = edge_index[1].astype(jnp.int32)
    eye = jnp.eye(H, dtype=jnp.float32)
    A_src = (att_src[:, :, None] * eye[:, None, :]).reshape(H * C, H)
    A_dst = (att_dst[:, :, None] * eye[:, None, :]).reshape(H * C, H)
    A = jnp.concatenate([A_src, A_dst], axis=1)

    xp, av, asrc_t, adst_t = _tc_pre(x, W, A)
    acc = _sc_scatter(xp, asrc_t, adst_t, src, dst)
    out = _tc_post(acc, xp, av, bias.reshape(1, D),
                   ln_scale.reshape(1, D), ln_bias.reshape(1, D))
    return out


# speedup vs baseline: 86.4006x; 2.7466x over previous
import functools

import jax
import jax.numpy as jnp
from jax import lax
from jax.experimental import pallas as pl
from jax.experimental.pallas import tpu as pltpu
from jax.experimental.pallas import tpu_sc as plsc

N = 10000
D = 128
H = 4
C = 32
E = 320000

NC = 2
NS = 16
NW = NC * NS
EPW = E // NW
K = 80
NBLK = EPW // K
ROW = 144
AW = 16
RPT = 624
TAIL = N - NS * RPT

_SQRT2 = 1.4142135623730951


def _pre_body(x_ref, w_ref, a_ref, xp_ref, av_ref, as_ref, ad_ref):
    xp = jnp.dot(x_ref[...], w_ref[...], preferred_element_type=jnp.float32)
    xp_ref[...] = xp
    av = jnp.dot(xp, a_ref[...], preferred_element_type=jnp.float32)
    av_ref[...] = av
    zpad = jnp.zeros((av.shape[0], AW - H), jnp.float32)
    as_ref[...] = jnp.concatenate([av[:, :H], zpad], axis=1)
    ad_ref[...] = jnp.concatenate([av[:, H:], zpad], axis=1)


def _tc_pre(x, W, A):
    RB = 1000
    return pl.pallas_call(
        _pre_body,
        grid=(N // RB,),
        in_specs=[
            pl.BlockSpec((RB, D), lambda i: (i, 0)),
            pl.BlockSpec((D, D), lambda i: (0, 0)),
            pl.BlockSpec((D, 2 * H), lambda i: (0, 0)),
        ],
        out_specs=[
            pl.BlockSpec((RB, D), lambda i: (i, 0)),
            pl.BlockSpec((RB, 2 * H), lambda i: (i, 0)),
            pl.BlockSpec((RB, AW), lambda i: (i, 0)),
            pl.BlockSpec((RB, AW), lambda i: (i, 0)),
        ],
        out_shape=[
            jax.ShapeDtypeStruct((N, D), jnp.float32),
            jax.ShapeDtypeStruct((N, 2 * H), jnp.float32),
            jax.ShapeDtypeStruct((N, AW), jnp.float32),
            jax.ShapeDtypeStruct((N, AW), jnp.float32),
        ],
    )(x, W, A)


def _sc_body(xp_hbm, as_hbm, ad_hbm, src_hbm, dst_hbm, out_hbm,
             src_v0, dst_v0, gs_v0, gd_v0, rows_v0,
             src_v1, dst_v1, gs_v1, gd_v1, rows_v1,
             sbuf_v, acc_sh,
             sa1, sa2, sa3, sb1, sb2, sb3):
    cid = lax.axis_index("c")
    sid = lax.axis_index("s")
    wid = cid * NS + sid
    base_e = wid * EPW
    slots = ((src_v0, dst_v0, gs_v0, gd_v0, rows_v0, sa1, sa2, sa3),
             (src_v1, dst_v1, gs_v1, gd_v1, rows_v1, sb1, sb2, sb3))

    def _zrow(j, _):
        for cc in range(ROW // 16):
            sbuf_v[j, pl.ds(cc * 16, 16)] = jnp.zeros((16,), jnp.float32)
        return 0
    lax.fori_loop(0, K, _zrow, 0)

    def _zacc(i, _):
        pltpu.sync_copy(sbuf_v.at[pl.ds(0, 48)],
                        acc_sh.at[pl.ds(sid * RPT + i * 48, 48)])
        return 0
    lax.fori_loop(0, RPT // 48, _zacc, 0)

    @pl.when(sid == NS - 1)
    def _ztail():
        pltpu.sync_copy(sbuf_v.at[pl.ds(0, TAIL)],
                        acc_sh.at[pl.ds(NS * RPT, TAIL)])
    plsc.subcore_barrier()

    lane = lax.broadcasted_iota(jnp.int32, (16,), 0)
    headmask = jnp.where(lane < H, 1.0, 0.0).astype(jnp.float32)

    def _issue(i, slot):
        srcv, dstv, gsv, gdv, rowsv, s1, s2, s3 = slot
        eb = base_e + i * K
        pltpu.sync_copy(src_hbm.at[pl.ds(eb, K)], srcv)
        pltpu.sync_copy(dst_hbm.at[pl.ds(eb, K)], dstv)
        pltpu.async_copy(xp_hbm.at[srcv], rowsv, s1)
        pltpu.async_copy(as_hbm.at[srcv], gsv, s2)
        pltpu.async_copy(ad_hbm.at[dstv], gdv, s3)

    def _finish_compute(slot):
        srcv, dstv, gsv, gdv, rowsv, s1, s2, s3 = slot
        pltpu.make_async_copy(xp_hbm.at[srcv], rowsv, s1).wait()
        pltpu.make_async_copy(as_hbm.at[srcv], gsv, s2).wait()
        pltpu.make_async_copy(ad_hbm.at[dstv], gdv, s3).wait()

        @plsc.parallel_loop(0, K, 1, unroll=4)
        def _edge(j):
            al = gsv[j, pl.ds(0, AW)] + gdv[j, pl.ds(0, AW)]
            al = jnp.maximum(al, 0.2 * al)
            w = jnp.exp(al)
            sbuf_v[j, pl.ds(D, 16)] = w * headmask
            for c8 in range(D // 16):
                wsc = w[c8 // 2]
                v = rowsv[j, pl.ds(c8 * 16, 16)]
                sbuf_v[j, pl.ds(c8 * 16, 16)] = v * wsc

        pltpu.sync_copy(sbuf_v, acc_sh.at[dstv], add=True)

    _issue(0, slots[0])

    def _pair(p, _):
        i0 = 2 * p
        _issue(i0 + 1, slots[1])
        _finish_compute(slots[0])
        _issue(i0 + 2, slots[0])
        _finish_compute(slots[1])
        return 0
    lax.fori_loop(0, (NBLK - 1) // 2, _pair, 0)
    _finish_compute(slots[0])
    plsc.subcore_barrier()

    pltpu.sync_copy(acc_sh.at[pl.ds(sid * RPT, RPT)],
                    out_hbm.at[cid, pl.ds(sid * RPT, RPT)])

    @pl.when(sid == NS - 1)
    def _ftail():
        pltpu.sync_copy(acc_sh.at[pl.ds(NS * RPT, TAIL)],
                        out_hbm.at[cid, pl.ds(NS * RPT, TAIL)])


def _sc_scatter(xp, asrc_t, adst_t, src, dst):
    mesh = plsc.VectorSubcoreMesh(core_axis_name="c", subcore_axis_name="s")
    f = functools.partial(
        pl.kernel,
        out_type=jax.ShapeDtypeStruct((NC, N, ROW), jnp.float32),
        mesh=mesh,
        compiler_params=pltpu.CompilerParams(needs_layout_passes=False,
                                             use_tc_tiling_on_sc=False),
        scratch_types=[
            pltpu.VMEM((K,), jnp.int32),
            pltpu.VMEM((K,), jnp.int32),
            pltpu.VMEM((K, AW), jnp.float32),
            pltpu.VMEM((K, AW), jnp.float32),
            pltpu.VMEM((K, D), jnp.float32),
            pltpu.VMEM((K,), jnp.int32),
            pltpu.VMEM((K,), jnp.int32),
            pltpu.VMEM((K, AW), jnp.float32),
            pltpu.VMEM((K, AW), jnp.float32),
            pltpu.VMEM((K, D), jnp.float32),
            pltpu.VMEM((K, ROW), jnp.float32),
            pltpu.VMEM_SHARED((N, ROW), jnp.float32),
            pltpu.SemaphoreType.DMA,
            pltpu.SemaphoreType.DMA,
            pltpu.SemaphoreType.DMA,
            pltpu.SemaphoreType.DMA,
            pltpu.SemaphoreType.DMA,
            pltpu.SemaphoreType.DMA,
        ],
    )(_sc_body)
    return f(xp, asrc_t, adst_t, src, dst)


def _post_body(acc_ref, xp_ref, av_ref, bias_ref, lns_ref, lnb_ref, o_ref):
    xp = xp_ref[...]
    av = av_ref[...]
    accs = acc_ref[0] + acc_ref[1]
    pieces = []
    for h in range(H):
        a_self = av[:, h:h + 1] + av[:, H + h:H + h + 1]
        w_self = jnp.exp(jnp.maximum(a_self, 0.2 * a_self))
        den = accs[:, D + h:D + h + 1] + w_self
        numh = accs[:, C * h:C * h + C] + xp[:, C * h:C * h + C] * w_self
        pieces.append(numh / den)
    hv = jnp.concatenate(pieces, axis=1) + bias_ref[...]
    hv = jnp.sqrt(hv * hv + 1e-12)
    mu = jnp.mean(hv, axis=1, keepdims=True)
    var = jnp.mean((hv - mu) ** 2, axis=1, keepdims=True)
    hv = (hv - mu) * lax.rsqrt(var + 1e-5) * lns_ref[...] + lnb_ref[...]
    o_ref[...] = hv * 0.5 * (1.0 + lax.erf(hv / _SQRT2))


def _tc_post(acc, xp, av, bias, ln_scale, ln_bias):
    RB = 1000
    return pl.pallas_call(
        _post_body,
        grid=(N // RB,),
        in_specs=[
            pl.BlockSpec((NC, RB, ROW), lambda i: (0, i, 0)),
            pl.BlockSpec((RB, D), lambda i: (i, 0)),
            pl.BlockSpec((RB, 2 * H), lambda i: (i, 0)),
            pl.BlockSpec((1, D), lambda i: (0, 0)),
            pl.BlockSpec((1, D), lambda i: (0, 0)),
            pl.BlockSpec((1, D), lambda i: (0, 0)),
        ],
        out_specs=pl.BlockSpec((RB, D), lambda i: (i, 0)),
        out_shape=jax.ShapeDtypeStruct((N, D), jnp.float32),
    )(acc, xp, av, bias, ln_scale, ln_bias)


def kernel(x, edge_index, W, att_src, att_dst, bias, phase, ln_scale, ln_bias):
    del phase
    src = edge_index[0].astype(jnp.int32)
    dst = edge_index[1].astype(jnp.int32)
    eye = jnp.eye(H, dtype=jnp.float32)
    A_src = (att_src[:, :, None] * eye[:, None, :]).reshape(H * C, H)
    A_dst = (att_dst[:, :, None] * eye[:, None, :]).reshape(H * C, H)
    A = jnp.concatenate([A_src, A_dst], axis=1)

    xp, av, asrc_t, adst_t = _tc_pre(x, W, A)
    acc = _sc_scatter(xp, asrc_t, adst_t, src, dst)
    out = _tc_post(acc, xp, av, bias.reshape(1, D),
                   ln_scale.reshape(1, D), ln_bias.reshape(1, D))
    return out
